# baseline (speedup 1.0000x reference)
import functools

import jax
import jax.numpy as jnp
from jax import lax
from jax.experimental import pallas as pl
from jax.experimental.pallas import tpu as pltpu
from jax.experimental.pallas import tpu_sc as plsc

_BLK = 128
_NBUF = 5


def _make_lookup(seq: int, batch: int, dim: int, nc: int, ns: int):
  nw = nc * ns
  assert batch == nw * _BLK and dim % 8 == 0 and seq % _NBUF == 0
  dt = dim // 8
  groups = seq // _NBUF

  mesh = plsc.VectorSubcoreMesh(core_axis_name="c", subcore_axis_name="s")

  @functools.partial(
      pl.kernel,
      out_type=jax.ShapeDtypeStruct((seq, dt, nw, 8, _BLK), jnp.float32),
      mesh=mesh,
      scratch_types=[
          pltpu.VMEM((seq, _BLK), jnp.int32),
          pltpu.VMEM((_NBUF, _BLK, dim), jnp.float32),
          pltpu.VMEM((_NBUF, dt, 8, _BLK), jnp.float32),
      ] + [pltpu.SemaphoreType.DMA] * (2 * _NBUF),
      compiler_params=pltpu.CompilerParams(
          use_tc_tiling_on_sc=False, needs_layout_passes=False),
  )
  def lookup_kernel(tok_hbm, table_hbm, out_hbm, idbuf, rows, outs, *sems):
    gsem = sems[:_NBUF]
    wsem = sems[_NBUF:]
    wid = lax.axis_index("s") * nc + lax.axis_index("c")

    pltpu.sync_copy(tok_hbm.at[:, pl.ds(wid * _BLK, _BLK)], idbuf)

    iota = lax.iota(jnp.int32, 16)
    ct_base = iota >> 3
    ci_vec = iota & 7

    def transpose(b):
      src = rows.at[b]
      dst = outs.at[b]

      @pl.loop(0, _BLK, unroll=4)
      def _(t):
        bi_vec = jnp.full((16,), t, jnp.int32)
        for c0 in range(dim // 16):
          val = src[t, pl.ds(c0 * 16, 16)]
          plsc.store_scatter(dst, [ct_base + c0 * 2, ci_vec, bi_vec], val)

    def gather(s, b):
      pltpu.async_copy(table_hbm.at[idbuf.at[s]], rows.at[b], gsem[b])

    def wait_gather(b):
      pltpu.make_async_copy(table_hbm.at[idbuf.at[0]], rows.at[b],
                            gsem[b]).wait()

    def put(s, b):
      pltpu.async_copy(outs.at[b], out_hbm.at[s, :, wid], wsem[b])

    def wait_put(b):
      pltpu.make_async_copy(outs.at[b], out_hbm.at[0, :, wid],
                            wsem[b]).wait()

    for b in range(_NBUF):
      gather(b, b)
    for b in range(_NBUF):
      wait_gather(b)
      transpose(b)
      put(b, b)
      gather(b + _NBUF, b)

    @pl.loop(1, groups - 1)
    def _(g):
      for b in range(_NBUF):
        s = g * _NBUF + b
        wait_gather(b)
        wait_put(b)
        transpose(b)
        put(s, b)
        gather(s + _NBUF, b)

    for b in range(_NBUF):
      s = seq - _NBUF + b
      wait_gather(b)
      wait_put(b)
      transpose(b)
      put(s, b)
    for b in range(_NBUF):
      wait_put(b)

  return lookup_kernel


def kernel(token_ids, weight):
  info = plsc.get_sparse_core_info()
  nc, ns = info.num_cores, info.num_subcores
  batch, seq = token_ids.shape
  dim = weight.shape[1]
  tok = token_ids.T.astype(jnp.int32)
  out5 = _make_lookup(seq, batch, dim, nc, ns)(tok, weight)
  return out5.transpose(2, 4, 0, 1, 3).reshape(batch, seq, dim)


# speedup vs baseline: 1.2837x; 1.2837x over previous
import functools

import jax
import jax.numpy as jnp
from jax import lax
from jax.experimental import pallas as pl
from jax.experimental.pallas import tpu as pltpu
from jax.experimental.pallas import tpu_sc as plsc

_BLK = 128
_NBUF = 5


def _make_lookup(seq: int, batch: int, dim: int, nc: int, ns: int):
  nw = nc * ns
  assert batch == nw * _BLK and dim % 8 == 0 and seq % _NBUF == 0
  dt = dim // 8
  groups = seq // _NBUF

  mesh = plsc.VectorSubcoreMesh(core_axis_name="c", subcore_axis_name="s")

  @functools.partial(
      pl.kernel,
      out_type=jax.ShapeDtypeStruct((seq, dt, nw, 8, _BLK), jnp.float32),
      mesh=mesh,
      scratch_types=[
          pltpu.VMEM((seq, _BLK), jnp.int32),
          pltpu.VMEM((_NBUF, _BLK, dim), jnp.float32),
          pltpu.VMEM((_NBUF, dt, 8, _BLK), jnp.float32),
      ] + [pltpu.SemaphoreType.DMA] * (2 * _NBUF),
      compiler_params=pltpu.CompilerParams(
          use_tc_tiling_on_sc=False, needs_layout_passes=False),
  )
  def lookup_kernel(tok_hbm, table_hbm, out_hbm, idbuf, rows, outs, *sems):
    gsem = sems[:_NBUF]
    wsem = sems[_NBUF:]
    wid = lax.axis_index("s") * nc + lax.axis_index("c")

    pltpu.sync_copy(tok_hbm.at[:, pl.ds(wid * _BLK, _BLK)], idbuf)

    iota = lax.iota(jnp.int32, 16)
    ct_base = iota >> 3
    ci_vec = iota & 7

    def transpose(b):
      src = rows.at[b]
      dst = outs.at[b]

      @plsc.parallel_loop(0, _BLK, unroll=4)
      def _(t):
        bi_vec = jnp.full((16,), t, jnp.int32)
        for c0 in range(dim // 16):
          val = src[t, pl.ds(c0 * 16, 16)]
          plsc.store_scatter(dst, [ct_base + c0 * 2, ci_vec, bi_vec], val)

    def gather(s, b):
      pltpu.async_copy(table_hbm.at[idbuf.at[s]], rows.at[b], gsem[b])

    def wait_gather(b):
      pltpu.make_async_copy(table_hbm.at[idbuf.at[0]], rows.at[b],
                            gsem[b]).wait()

    def put(s, b):
      pltpu.async_copy(outs.at[b], out_hbm.at[s, :, wid], wsem[b])

    def wait_put(b):
      pltpu.make_async_copy(outs.at[b], out_hbm.at[0, :, wid],
                            wsem[b]).wait()

    for b in range(_NBUF):
      gather(b, b)
    for b in range(_NBUF):
      wait_gather(b)
      transpose(b)
      put(b, b)
      gather(b + _NBUF, b)

    @pl.loop(1, groups - 1)
    def _(g):
      for b in range(_NBUF):
        s = g * _NBUF + b
        wait_gather(b)
        wait_put(b)
        transpose(b)
        put(s, b)
        gather(s + _NBUF, b)

    for b in range(_NBUF):
      s = seq - _NBUF + b
      wait_gather(b)
      wait_put(b)
      transpose(b)
      put(s, b)
    for b in range(_NBUF):
      wait_put(b)

  return lookup_kernel


def kernel(token_ids, weight):
  info = plsc.get_sparse_core_info()
  nc, ns = info.num_cores, info.num_subcores
  batch, seq = token_ids.shape
  dim = weight.shape[1]
  tok = token_ids.T.astype(jnp.int32)
  out5 = _make_lookup(seq, batch, dim, nc, ns)(tok, weight)
  return out5.transpose(2, 4, 0, 1, 3).reshape(batch, seq, dim)


# speedup vs baseline: 2.7817x; 2.1669x over previous
import functools

import jax
import jax.numpy as jnp
from jax import lax
from jax.experimental import pallas as pl
from jax.experimental.pallas import tpu as pltpu
from jax.experimental.pallas import tpu_sc as plsc

_BLK = 128
_NBUF = 5


def _make_lookup(seq: int, batch: int, dim: int, nc: int, ns: int):
  nw = nc * ns
  assert batch == nw * _BLK and dim % 8 == 0 and seq % _NBUF == 0
  dt = dim // 8
  groups = seq // _NBUF

  mesh = plsc.VectorSubcoreMesh(core_axis_name="c", subcore_axis_name="s")

  @functools.partial(
      pl.kernel,
      out_type=jax.ShapeDtypeStruct((seq, dt, nw, 8, _BLK), jnp.float32),
      mesh=mesh,
      scratch_types=[
          pltpu.VMEM((seq, _BLK), jnp.int32),
          pltpu.VMEM((_NBUF, _BLK, dim), jnp.float32),
          pltpu.VMEM((_NBUF, dt, 8, _BLK + 1), jnp.float32),
      ] + [pltpu.SemaphoreType.DMA] * (2 * _NBUF),
      compiler_params=pltpu.CompilerParams(
          use_tc_tiling_on_sc=False, needs_layout_passes=False),
  )
  def lookup_kernel(tok_hbm, table_hbm, out_hbm, idbuf, rows, outs, *sems):
    gsem = sems[:_NBUF]
    wsem = sems[_NBUF:]
    wid = lax.axis_index("s") * nc + lax.axis_index("c")

    pltpu.sync_copy(tok_hbm.at[:, pl.ds(wid * _BLK, _BLK)], idbuf)

    iota = lax.iota(jnp.int32, 16)
    ct_base = iota >> 3
    ci_vec = iota & 7

    def transpose(b):
      src = rows.at[b]
      dst = outs.at[b]

      @plsc.parallel_loop(0, _BLK, unroll=4)
      def _(t):
        bi_vec = jnp.full((16,), t, jnp.int32)
        for c0 in range(dim // 16):
          val = src[t, pl.ds(c0 * 16, 16)]
          plsc.store_scatter(dst, [ct_base + c0 * 2, ci_vec, bi_vec], val)

    def gather(s, b):
      pltpu.async_copy(table_hbm.at[idbuf.at[s]], rows.at[b], gsem[b])

    def wait_gather(b):
      pltpu.make_async_copy(table_hbm.at[idbuf.at[0]], rows.at[b],
                            gsem[b]).wait()

    def put(s, b):
      pltpu.async_copy(outs.at[b, :, :, pl.ds(0, _BLK)],
                       out_hbm.at[s, :, wid], wsem[b])

    def wait_put(b):
      pltpu.make_async_copy(outs.at[b, :, :, pl.ds(0, _BLK)],
                            out_hbm.at[0, :, wid], wsem[b]).wait()

    for b in range(_NBUF):
      gather(b, b)
    for b in range(_NBUF):
      wait_gather(b)
      transpose(b)
      put(b, b)
      gather(b + _NBUF, b)

    @pl.loop(1, groups - 1)
    def _(g):
      for b in range(_NBUF):
        s = g * _NBUF + b
        wait_gather(b)
        wait_put(b)
        transpose(b)
        put(s, b)
        gather(s + _NBUF, b)

    for b in range(_NBUF):
      s = seq - _NBUF + b
      wait_gather(b)
      wait_put(b)
      transpose(b)
      put(s, b)
    for b in range(_NBUF):
      wait_put(b)

  return lookup_kernel


def kernel(token_ids, weight):
  info = plsc.get_sparse_core_info()
  nc, ns = info.num_cores, info.num_subcores
  batch, seq = token_ids.shape
  dim = weight.shape[1]
  tok = token_ids.T.astype(jnp.int32)
  out5 = _make_lookup(seq, batch, dim, nc, ns)(tok, weight)
  return out5.transpose(2, 4, 0, 1, 3).reshape(batch, seq, dim)
